# baseline (speedup 1.0000x reference)
import functools

import jax
import jax.numpy as jnp
from jax import lax
from jax.experimental import pallas as pl
from jax.experimental.pallas import tpu as pltpu
from jax.experimental.pallas import tpu_sc as plsc

NUM_BUCKETS = 100000
EMBED_DIM = 128
BATCH = 16384

NUM_CORES = 2
NUM_SUBCORES = 16
NUM_WORKERS = NUM_CORES * NUM_SUBCORES
B_PER_W = BATCH // NUM_WORKERS
LANES = 16
CHUNKS = ((0, 128), (128, 128), (256, 128), (384, 96), (480, 32))


def _sc_embed_lookup(x_hbm, table_hbm, out_hbm, idx_v, rows_v, sem,
                     store_sem, idx_sem):
    wid = lax.axis_index("s") * NUM_CORES + lax.axis_index("c")
    base = wid * B_PER_W

    pltpu.sync_copy(x_hbm.at[wid, pl.ds(0, CHUNKS[0][1])],
                    idx_v.at[pl.ds(0, CHUNKS[0][1])])
    rest = CHUNKS[1][0]
    idx_cp = pltpu.async_copy(x_hbm.at[wid, pl.ds(rest, B_PER_W - rest)],
                              idx_v.at[pl.ds(rest, B_PER_W - rest)], idx_sem)

    gathers = []
    for c, (off, size) in enumerate(CHUNKS):
        if c == 1:
            idx_cp.wait()
        for i in range(size // LANES):
            v = idx_v[pl.ds(off + i * LANES, LANES)]
            q = (v.astype(jnp.float32) * jnp.float32(1.0 / NUM_BUCKETS)
                 ).astype(jnp.int32)
            r = v - q * NUM_BUCKETS
            r = jnp.where(r < 0, r + NUM_BUCKETS, r)
            r = jnp.where(r >= NUM_BUCKETS, r - NUM_BUCKETS, r)
            idx_v[pl.ds(off + i * LANES, LANES)] = r
        gathers.append(pltpu.async_copy(
            table_hbm.at[idx_v.at[pl.ds(off, size)]],
            rows_v.at[pl.ds(off, size)],
            sem))

    stores = []
    for c, (off, size) in enumerate(CHUNKS):
        gathers[c].wait()
        stores.append(pltpu.async_copy(
            rows_v.at[pl.ds(off, size)],
            out_hbm.at[pl.ds(base + off, size)],
            store_sem))
    for cp in stores:
        cp.wait()


@jax.jit
def kernel(x, emb_weight):
    x2 = x.astype(jnp.int32).reshape(NUM_WORKERS, B_PER_W)
    mesh = plsc.VectorSubcoreMesh(
        core_axis_name="c", subcore_axis_name="s",
        num_cores=NUM_CORES, num_subcores=NUM_SUBCORES)
    f = functools.partial(
        pl.kernel,
        out_type=jax.ShapeDtypeStruct((BATCH, EMBED_DIM), jnp.float32),
        mesh=mesh,
        scratch_types=[
            pltpu.VMEM((B_PER_W,), jnp.int32),
            pltpu.VMEM((B_PER_W, EMBED_DIM), jnp.float32),
            pltpu.SemaphoreType.DMA,
            pltpu.SemaphoreType.DMA,
            pltpu.SemaphoreType.DMA,
        ],
    )(_sc_embed_lookup)
    return f(x2, emb_weight)


# speedup vs baseline: 1.0014x; 1.0014x over previous
import functools

import jax
import jax.numpy as jnp
from jax import lax
from jax.experimental import pallas as pl
from jax.experimental.pallas import tpu as pltpu
from jax.experimental.pallas import tpu_sc as plsc

NUM_BUCKETS = 100000
EMBED_DIM = 128
BATCH = 16384

NUM_CORES = 2
NUM_SUBCORES = 16
NUM_WORKERS = NUM_CORES * NUM_SUBCORES
B_PER_W = BATCH // NUM_WORKERS
LANES = 16
CHUNKS = ((0, 128), (128, 128), (256, 128), (384, 96), (480, 32))


def _sc_embed_lookup(x_hbm, table_hbm, out_hbm, idx_v, rows_v, sem,
                     store_sem, idx_sem):
    wid = lax.axis_index("s") * NUM_CORES + lax.axis_index("c")
    base = wid * B_PER_W

    pltpu.sync_copy(x_hbm.at[pl.ds(base, CHUNKS[0][1])],
                    idx_v.at[pl.ds(0, CHUNKS[0][1])])
    rest = CHUNKS[1][0]
    idx_cp = pltpu.async_copy(x_hbm.at[pl.ds(base + rest, B_PER_W - rest)],
                              idx_v.at[pl.ds(rest, B_PER_W - rest)], idx_sem)

    gathers = []
    for c, (off, size) in enumerate(CHUNKS):
        if c == 1:
            idx_cp.wait()
        for i in range(size // LANES):
            v = idx_v[pl.ds(off + i * LANES, LANES)]
            q = (v.astype(jnp.float32) * jnp.float32(1.0 / NUM_BUCKETS)
                 ).astype(jnp.int32)
            r = v - q * NUM_BUCKETS
            r = jnp.where(r < 0, r + NUM_BUCKETS, r)
            r = jnp.where(r >= NUM_BUCKETS, r - NUM_BUCKETS, r)
            idx_v[pl.ds(off + i * LANES, LANES)] = r
        gathers.append(pltpu.async_copy(
            table_hbm.at[idx_v.at[pl.ds(off, size)]],
            rows_v.at[pl.ds(off, size)],
            sem))

    stores = []
    for c, (off, size) in enumerate(CHUNKS):
        gathers[c].wait()
        stores.append(pltpu.async_copy(
            rows_v.at[pl.ds(off, size)],
            out_hbm.at[pl.ds(base + off, size)],
            store_sem))
    for cp in stores:
        cp.wait()


@jax.jit
def kernel(x, emb_weight):
    x1 = x.astype(jnp.int32)
    mesh = plsc.VectorSubcoreMesh(
        core_axis_name="c", subcore_axis_name="s",
        num_cores=NUM_CORES, num_subcores=NUM_SUBCORES)
    f = functools.partial(
        pl.kernel,
        out_type=jax.ShapeDtypeStruct((BATCH, EMBED_DIM), jnp.float32),
        mesh=mesh,
        scratch_types=[
            pltpu.VMEM((B_PER_W,), jnp.int32),
            pltpu.VMEM((B_PER_W, EMBED_DIM), jnp.float32),
            pltpu.SemaphoreType.DMA,
            pltpu.SemaphoreType.DMA,
            pltpu.SemaphoreType.DMA,
        ],
    )(_sc_embed_lookup)
    return f(x1, emb_weight)
